# speedup vs baseline: 1.4919x; 1.4919x over previous
import jax
import jax.numpy as jnp
from jax import lax
from jax.experimental import pallas as pl
from jax.experimental.pallas import tpu as pltpu
from jax.experimental.pallas import tpu_sc as plsc

VOCAB = 49408
EMBED = 768
MAX_LEN = 77
BATCH = 1024

N = BATCH * MAX_LEN
NC, NS = 2, 16
NW = NC * NS
R = N // NW
C = 56
NCH = R // C
SUB = EMBED // 16


def _sc_body(tok_hbm, posn_hbm, ttab_hbm, ptab_hbm, out_hbm,
             idx_t, idx_p, buf_t, buf_p, sem_t, sem_p):
    wid = lax.axis_index("s") * NC + lax.axis_index("c")
    base = pl.multiple_of(wid * R, 8)
    pltpu.sync_copy(tok_hbm.at[pl.ds(base, R)], idx_t)
    pltpu.sync_copy(posn_hbm.at[pl.ds(base, R)], idx_p)

    def chunk(c, carry):
        off = pl.multiple_of(c * C, 8)
        cp_t = pltpu.async_copy(ttab_hbm.at[idx_t.at[pl.ds(off, C)]], buf_t, sem_t)
        cp_p = pltpu.async_copy(ptab_hbm.at[idx_p.at[pl.ds(off, C)]], buf_p, sem_p)
        cp_t.wait()
        cp_p.wait()

        def row(r, rcarry):
            for j in range(SUB):
                s = pl.ds(j * 16, 16)
                buf_t[r, s] = buf_t[r, s] + buf_p[r, s]
            return rcarry

        lax.fori_loop(0, C, row, 0)
        pltpu.sync_copy(buf_t, out_hbm.at[pl.ds(base + off, C)])
        return carry

    lax.fori_loop(0, NCH, chunk, 0)


def kernel(tokens, positions, token_table, pos_table):
    tok = tokens.reshape(N)
    posn = positions.reshape(N)
    mesh = plsc.VectorSubcoreMesh(
        core_axis_name="c", subcore_axis_name="s",
        num_cores=NC, num_subcores=NS)
    out = pl.kernel(
        _sc_body,
        out_type=jax.ShapeDtypeStruct((N, EMBED), jnp.float32),
        mesh=mesh,
        scratch_types=[
            pltpu.VMEM((R,), jnp.int32),
            pltpu.VMEM((R,), jnp.int32),
            pltpu.VMEM((C, EMBED), jnp.float32),
            pltpu.VMEM((C, EMBED), jnp.float32),
            pltpu.SemaphoreType.DMA,
            pltpu.SemaphoreType.DMA,
        ],
    )(tok, posn, token_table, pos_table)
    return out.reshape(BATCH, MAX_LEN, EMBED)


# speedup vs baseline: 1.5883x; 1.0646x over previous
import jax
import jax.numpy as jnp
from jax import lax
from jax.experimental import pallas as pl
from jax.experimental.pallas import tpu as pltpu
from jax.experimental.pallas import tpu_sc as plsc

VOCAB = 49408
EMBED = 768
MAX_LEN = 77
BATCH = 1024

N = BATCH * MAX_LEN
NC, NS = 2, 16
NW = NC * NS
R = N // NW
C = 16
NCH = R // C
SUB = EMBED // 16


def _sc_body(tok_hbm, posn_hbm, ttab_hbm, ptab_hbm, out_hbm,
             idx_t, idx_p, bt0, bt1, bp0, bp1, bo0, bo1,
             st0, st1, sp0, sp1, so0, so1):
    bt, bp, bo = (bt0, bt1), (bp0, bp1), (bo0, bo1)
    st, sp, so = (st0, st1), (sp0, sp1), (so0, so1)

    wid = lax.axis_index("s") * NC + lax.axis_index("c")
    base = pl.multiple_of(wid * R, 8)
    pltpu.sync_copy(tok_hbm.at[pl.ds(base, R)], idx_t)
    pltpu.sync_copy(posn_hbm.at[pl.ds(base, R)], idx_p)

    def issue_gather(i, par):
        off = pl.multiple_of(i * C, 8)
        pltpu.async_copy(ttab_hbm.at[idx_t.at[pl.ds(off, C)]], bt[par], st[par])
        pltpu.async_copy(ptab_hbm.at[idx_p.at[pl.ds(off, C)]], bp[par], sp[par])

    def wait_gather(par):
        pltpu.make_async_copy(
            ttab_hbm.at[idx_t.at[pl.ds(0, C)]], bt[par], st[par]).wait()
        pltpu.make_async_copy(
            ptab_hbm.at[idx_p.at[pl.ds(0, C)]], bp[par], sp[par]).wait()

    def wait_out(par):
        pltpu.make_async_copy(
            bo[par], out_hbm.at[pl.ds(0, C)], so[par]).wait()

    def step(i, k, par):
        wait_gather(par)

        @pl.when(k >= 1)
        def _():
            wait_out(par)

        def row(r, rcarry):
            for j in range(SUB):
                s = pl.ds(j * 16, 16)
                bo[par][r, s] = bt[par][r, s] + bp[par][r, s]
            return rcarry

        lax.fori_loop(0, C, row, 0)
        off = pl.multiple_of(i * C, 8)
        pltpu.async_copy(bo[par], out_hbm.at[pl.ds(base + off, C)], so[par])

        @pl.when(k < (NCH // 2) - 1)
        def _():
            issue_gather(i + 2, par)

    issue_gather(0, 0)
    issue_gather(1, 1)

    def pair(k, carry):
        step(2 * k, k, 0)
        step(2 * k + 1, k, 1)
        return carry

    lax.fori_loop(0, NCH // 2, pair, 0)
    wait_out(0)
    wait_out(1)


def kernel(tokens, positions, token_table, pos_table):
    tok = tokens.reshape(N)
    posn = positions.reshape(N)
    mesh = plsc.VectorSubcoreMesh(
        core_axis_name="c", subcore_axis_name="s",
        num_cores=NC, num_subcores=NS)
    out = pl.kernel(
        _sc_body,
        out_type=jax.ShapeDtypeStruct((N, EMBED), jnp.float32),
        mesh=mesh,
        scratch_types=[
            pltpu.VMEM((R,), jnp.int32),
            pltpu.VMEM((R,), jnp.int32),
            pltpu.VMEM((C, EMBED), jnp.float32),
            pltpu.VMEM((C, EMBED), jnp.float32),
            pltpu.VMEM((C, EMBED), jnp.float32),
            pltpu.VMEM((C, EMBED), jnp.float32),
            pltpu.VMEM((C, EMBED), jnp.float32),
            pltpu.VMEM((C, EMBED), jnp.float32),
            pltpu.SemaphoreType.DMA,
            pltpu.SemaphoreType.DMA,
            pltpu.SemaphoreType.DMA,
            pltpu.SemaphoreType.DMA,
            pltpu.SemaphoreType.DMA,
            pltpu.SemaphoreType.DMA,
        ],
    )(tok, posn, token_table, pos_table)
    return out.reshape(BATCH, MAX_LEN, EMBED)


# speedup vs baseline: 1.6459x; 1.0363x over previous
import jax
import jax.numpy as jnp
from jax import lax
from jax.experimental import pallas as pl
from jax.experimental.pallas import tpu as pltpu
from jax.experimental.pallas import tpu_sc as plsc

VOCAB = 49408
EMBED = 768
MAX_LEN = 77
LPAD = 80
BATCH = 1024

NC, NS = 2, 16
NW = NC * NS
BW = BATCH // NW
RW = BW * LPAD
CR = 16
SUBS = (16, 16, 16, 16, 13)
NSUB = len(SUBS)
SUB = EMBED // 16


def _sc_body(tok_hbm, posn_hbm, ttab_hbm, ptab_hbm, out_hbm,
             idx_t, idx_p, bt0, bt1, bp0, bp1, bo,
             st0, st1, sp0, sp1, so):
    bt, bp = (bt0, bt1), (bp0, bp1)
    st, sp = (st0, st1), (sp0, sp1)

    wid = lax.axis_index("s") * NC + lax.axis_index("c")
    base = pl.multiple_of(wid * RW, 8)
    pltpu.sync_copy(tok_hbm.at[pl.ds(base, RW)], idx_t)
    pltpu.sync_copy(posn_hbm.at[pl.ds(base, RW)], idx_p)

    b0 = wid * BW

    def issue_gather(jb, j, par):
        off = pl.multiple_of(jb * LPAD + j * CR, 8)
        pltpu.async_copy(ttab_hbm.at[idx_t.at[pl.ds(off, CR)]],
                         bt[par], st[par])
        pltpu.async_copy(ptab_hbm.at[idx_p.at[pl.ds(off, CR)]],
                         bp[par], sp[par])

    def wait_gather(par):
        pltpu.make_async_copy(ttab_hbm.at[idx_t.at[pl.ds(0, CR)]],
                              bt[par], st[par]).wait()
        pltpu.make_async_copy(ptab_hbm.at[idx_p.at[pl.ds(0, CR)]],
                              bp[par], sp[par]).wait()

    def wait_out():
        pltpu.make_async_copy(bo, out_hbm.at[0], so).wait()

    def step(k, jj):
        second = jj >= NSUB
        j = jj % NSUB
        par = jj % 2
        n = SUBS[j]
        jb = 2 * k + (1 if second else 0)

        wait_gather(par)
        if jj < 2 * NSUB - 1:
            if j + 1 < NSUB:
                issue_gather(jb, j + 1, par ^ 1)
            else:
                issue_gather(jb + 1, 0, par ^ 1)
        else:
            def do_issue():
                issue_gather(jb + 1, 0, par ^ 1)
            pl.when(k < (BW // 2) - 1)(do_issue)

        if j == 0:
            if second:
                wait_out()
            else:
                pl.when(k >= 1)(wait_out)

        def row(r, rcarry):
            for g in range(SUB):
                s = pl.ds(g * 16, 16)
                bo[j * CR + r, s] = bt[par][r, s] + bp[par][r, s]
            return rcarry

        lax.fori_loop(0, n, row, 0)

        if j == NSUB - 1:
            pltpu.async_copy(bo, out_hbm.at[b0 + jb], so)

    issue_gather(0, 0, 0)

    def pair(k, carry):
        for jj in range(2 * NSUB):
            step(k, jj)
        return carry

    lax.fori_loop(0, BW // 2, pair, 0)
    wait_out()


def kernel(tokens, positions, token_table, pos_table):
    tok = jnp.pad(tokens, ((0, 0), (0, LPAD - MAX_LEN))).reshape(BATCH * LPAD)
    posn = jnp.pad(positions, ((0, 0), (0, LPAD - MAX_LEN))).reshape(BATCH * LPAD)
    mesh = plsc.VectorSubcoreMesh(
        core_axis_name="c", subcore_axis_name="s",
        num_cores=NC, num_subcores=NS)
    return pl.kernel(
        _sc_body,
        out_type=jax.ShapeDtypeStruct((BATCH, MAX_LEN, EMBED), jnp.float32),
        mesh=mesh,
        scratch_types=[
            pltpu.VMEM((RW,), jnp.int32),
            pltpu.VMEM((RW,), jnp.int32),
            pltpu.VMEM((CR, EMBED), jnp.float32),
            pltpu.VMEM((CR, EMBED), jnp.float32),
            pltpu.VMEM((CR, EMBED), jnp.float32),
            pltpu.VMEM((CR, EMBED), jnp.float32),
            pltpu.VMEM((MAX_LEN, EMBED), jnp.float32),
            pltpu.SemaphoreType.DMA,
            pltpu.SemaphoreType.DMA,
            pltpu.SemaphoreType.DMA,
            pltpu.SemaphoreType.DMA,
            pltpu.SemaphoreType.DMA,
        ],
    )(tok, posn, token_table, pos_table)


# speedup vs baseline: 1.6907x; 1.0272x over previous
import jax
import jax.numpy as jnp
from jax import lax
from jax.experimental import pallas as pl
from jax.experimental.pallas import tpu as pltpu
from jax.experimental.pallas import tpu_sc as plsc

VOCAB = 49408
EMBED = 768
MAX_LEN = 77
BATCH = 1024
ROWS = BATCH * MAX_LEN

NC, NS = 2, 16
NW = NC * NS
RW = ROWS // NW
CR = 16
NCH = RW // CR
NB = 4
SUB = EMBED // 16


def _sc_body(tok_hbm, posn_hbm, ttab_hbm, ptab_hbm, out_hbm,
             idx_t, idx_p, ptab_v, b0, b1, b2, b3,
             g0, g1, g2, g3, o0, o1, o2, o3):
    bt = (b0, b1, b2, b3)
    gs = (g0, g1, g2, g3)
    os_ = (o0, o1, o2, o3)

    wid = lax.axis_index("s") * NC + lax.axis_index("c")
    base = pl.multiple_of(wid * RW, 8)
    pltpu.sync_copy(tok_hbm.at[pl.ds(base, RW)], idx_t)
    pltpu.sync_copy(posn_hbm.at[pl.ds(base, RW)], idx_p)
    pltpu.sync_copy(ptab_hbm, ptab_v)

    def issue_gather(c, par):
        off = pl.multiple_of(c * CR, 8)
        pltpu.async_copy(ttab_hbm.at[idx_t.at[pl.ds(off, CR)]],
                         bt[par], gs[par])

    def wait_gather(par):
        pltpu.make_async_copy(ttab_hbm.at[idx_t.at[pl.ds(0, CR)]],
                              bt[par], gs[par]).wait()

    def issue_out(c, par):
        dst = pl.multiple_of(base + c * CR, 8)
        pltpu.async_copy(bt[par], out_hbm.at[pl.ds(dst, CR)], os_[par])

    def wait_out(par):
        pltpu.make_async_copy(bt[par], out_hbm.at[pl.ds(0, CR)],
                              os_[par]).wait()

    cols = [jnp.arange(16, dtype=jnp.int32) + 16 * g for g in range(SUB)]

    def compute(c, par):
        coff = pl.multiple_of(c * CR, 8)
        p_vec = idx_p[pl.ds(coff, CR)]

        def row(r, carry):
            pb = p_vec.at[jnp.full((16,), r, dtype=jnp.int32)].get(
                mode="promise_in_bounds")
            for g0 in range(0, SUB, 8):
                pvs = [plsc.load_gather(ptab_v, [pb + cols[g]])
                       for g in range(g0, g0 + 8)]
                for g in range(g0, g0 + 8):
                    plsc.addupdate(bt[par].at[r, pl.ds(16 * g, 16)],
                                   pvs[g - g0])
            return carry

        lax.fori_loop(0, CR, row, 0)

    issue_gather(0, 0)
    issue_gather(1, 1)

    def main(k, carry):
        for jj in range(NB):
            c = NB * k + jj
            par = jj
            pnext = (jj + 2) % NB
            wait_gather(par)
            pl.when(c >= 2)(lambda: wait_out(pnext))
            issue_gather(c + 2, pnext)
            compute(c, par)
            issue_out(c, par)
        return carry

    lax.fori_loop(0, (NCH - 2) // NB, main, 0)

    for c in (NCH - 2, NCH - 1):
        par = c % NB
        wait_gather(par)
        compute(c, par)
        issue_out(c, par)

    for c in range(NCH - NB, NCH):
        wait_out(c % NB)


def kernel(tokens, positions, token_table, pos_table):
    tok = tokens.reshape(ROWS)
    posn = positions.reshape(ROWS) * EMBED
    ptab = pos_table.reshape(MAX_LEN * EMBED)
    mesh = plsc.VectorSubcoreMesh(
        core_axis_name="c", subcore_axis_name="s",
        num_cores=NC, num_subcores=NS)
    out = pl.kernel(
        _sc_body,
        out_type=jax.ShapeDtypeStruct((ROWS, EMBED), jnp.float32),
        mesh=mesh,
        compiler_params=pltpu.CompilerParams(
            use_tc_tiling_on_sc=False, needs_layout_passes=False),
        scratch_types=[
            pltpu.VMEM((RW,), jnp.int32),
            pltpu.VMEM((RW,), jnp.int32),
            pltpu.VMEM((MAX_LEN * EMBED,), jnp.float32),
            pltpu.VMEM((CR, EMBED), jnp.float32),
            pltpu.VMEM((CR, EMBED), jnp.float32),
            pltpu.VMEM((CR, EMBED), jnp.float32),
            pltpu.VMEM((CR, EMBED), jnp.float32),
            pltpu.SemaphoreType.DMA,
            pltpu.SemaphoreType.DMA,
            pltpu.SemaphoreType.DMA,
            pltpu.SemaphoreType.DMA,
            pltpu.SemaphoreType.DMA,
            pltpu.SemaphoreType.DMA,
            pltpu.SemaphoreType.DMA,
            pltpu.SemaphoreType.DMA,
        ],
    )(tok, posn, token_table, ptab)
    return out.reshape(BATCH, MAX_LEN, EMBED)


# speedup vs baseline: 2.3848x; 1.4106x over previous
import jax
import jax.numpy as jnp
from jax import lax
from jax.experimental import pallas as pl
from jax.experimental.pallas import tpu as pltpu
from jax.experimental.pallas import tpu_sc as plsc

VOCAB = 49408
EMBED = 768
MAX_LEN = 77
BATCH = 1024
LPAD = 80

NC, NS = 2, 16
NW = NC * NS
EW = BATCH // NW
CR = 8
NFULL = MAX_LEN // CR
TAIL = MAX_LEN - NFULL * CR
SUB = EMBED // 16


def _sc_body(tok_hbm, posn_hbm, ttab_hbm, ptab_hbm, out_hbm,
             it0, it1, idx_p, ptab_v, bo, tb,
             si0, si1, so,
             c0, c1, c2, c3, c4, c5, c6, c7, c8, c9):
    it = (it0, it1)
    si = (si0, si1)
    cs = (c0, c1, c2, c3, c4, c5, c6, c7, c8, c9)

    wid = lax.axis_index("s") * NC + lax.axis_index("c")
    pltpu.sync_copy(posn_hbm.at[pl.ds(pl.multiple_of(wid * EW * LPAD, 8),
                                      EW * LPAD)],
                    idx_p.at[pl.ds(0, EW * LPAD)])
    pltpu.sync_copy(ptab_hbm, ptab_v)

    ent0 = wid * EW

    def issue_idx(e, par):
        off = pl.multiple_of((ent0 + e) * LPAD, 8)
        pltpu.async_copy(tok_hbm.at[pl.ds(off, LPAD)], it[par], si[par])

    def wait_idx(par):
        pltpu.make_async_copy(tok_hbm.at[pl.ds(0, LPAD)],
                              it[par], si[par]).wait()

    def issue_gathers(par):
        for j in range(NFULL):
            pltpu.async_copy(
                ttab_hbm.at[it[par].at[pl.ds(j * CR, CR)]],
                bo.at[pl.ds(j * CR, CR)], cs[j])
        pltpu.async_copy(
            ttab_hbm.at[it[par].at[pl.ds(NFULL * CR, TAIL)]], tb, cs[NFULL])

    def wait_gather(par, j):
        if j < NFULL:
            pltpu.make_async_copy(ttab_hbm.at[it[par].at[pl.ds(0, CR)]],
                                  bo.at[pl.ds(0, CR)], cs[j]).wait()
        else:
            pltpu.make_async_copy(ttab_hbm.at[it[par].at[pl.ds(0, TAIL)]],
                                  tb, cs[j]).wait()

    def issue_out(e):
        pltpu.async_copy(bo, out_hbm.at[ent0 + e], so)

    def wait_out():
        pltpu.make_async_copy(bo, out_hbm.at[0], so).wait()

    cols = [jnp.arange(16, dtype=jnp.int32) + 16 * g for g in range(SUB)]

    def compute_chunk(e, j):
        n = CR if j < NFULL else TAIL
        coff = pl.multiple_of(e * LPAD + j * CR, 8)
        p_vec = idx_p[pl.ds(coff, 16)]

        def row(r, carry):
            pb = p_vec.at[jnp.full((16,), r, dtype=jnp.int32)].get(
                mode="promise_in_bounds")
            for q in range(0, SUB, 8):
                pvs = [plsc.load_gather(ptab_v, [pb + cols[g]])
                       for g in range(q, q + 8)]
                for g in range(q, q + 8):
                    s = pl.ds(16 * g, 16)
                    if j < NFULL:
                        plsc.addupdate(bo.at[j * CR + r, s], pvs[g - q])
                    else:
                        bo[j * CR + r, s] = tb[r, s] + pvs[g - q]
            return carry

        lax.fori_loop(0, n, row, 0)

    issue_idx(0, 0)

    def entry(e, carry):
        for par in range(2):
            ee = 2 * e + par
            wait_idx(par)
            pl.when(ee >= 1)(wait_out)
            issue_gathers(par)
            pl.when(ee < EW - 1)(lambda: issue_idx(ee + 1, par ^ 1))
            for j in range(NFULL + 1):
                wait_gather(par, j)
                compute_chunk(ee, j)
            issue_out(ee)
        return carry

    lax.fori_loop(0, EW // 2, entry, 0)
    wait_out()


def kernel(tokens, positions, token_table, pos_table):
    pad = ((0, 0), (0, LPAD - MAX_LEN))
    tok = jnp.pad(tokens, pad).reshape(BATCH * LPAD)
    posn = jnp.pad(positions, pad).reshape(BATCH * LPAD) * EMBED
    ptab = pos_table.reshape(MAX_LEN * EMBED)
    mesh = plsc.VectorSubcoreMesh(
        core_axis_name="c", subcore_axis_name="s",
        num_cores=NC, num_subcores=NS)
    return pl.kernel(
        _sc_body,
        out_type=jax.ShapeDtypeStruct((BATCH, MAX_LEN, EMBED), jnp.float32),
        mesh=mesh,
        compiler_params=pltpu.CompilerParams(
            use_tc_tiling_on_sc=True, needs_layout_passes=False),
        scratch_types=[
            pltpu.VMEM((LPAD,), jnp.int32),
            pltpu.VMEM((LPAD,), jnp.int32),
            pltpu.VMEM((EW * LPAD + 16,), jnp.int32),
            pltpu.VMEM((MAX_LEN * EMBED,), jnp.float32),
            pltpu.VMEM((MAX_LEN, EMBED), jnp.float32),
            pltpu.VMEM((TAIL, EMBED), jnp.float32),
            pltpu.SemaphoreType.DMA,
            pltpu.SemaphoreType.DMA,
            pltpu.SemaphoreType.DMA,
            pltpu.SemaphoreType.DMA,
            pltpu.SemaphoreType.DMA,
            pltpu.SemaphoreType.DMA,
            pltpu.SemaphoreType.DMA,
            pltpu.SemaphoreType.DMA,
            pltpu.SemaphoreType.DMA,
            pltpu.SemaphoreType.DMA,
            pltpu.SemaphoreType.DMA,
            pltpu.SemaphoreType.DMA,
            pltpu.SemaphoreType.DMA,
        ],
    )(tok, posn, token_table, ptab)
